# baseline (speedup 1.0000x reference)
import functools

import jax
import jax.numpy as jnp
import numpy as np
from jax import lax
from jax.experimental import pallas as pl
from jax.experimental.pallas import tpu as pltpu
from jax.experimental.pallas import tpu_sc as plsc

_B = 512
_ED = 128
_NTYPE = 3
_NSCHEMA = 2
_TOT = 18
_NROWS = 3

_NC = 2
_NS = 16
_NW = _NC * _NS
_RPW = _B // _NW

_OFF0 = (0, 16)
_OFF1 = (32, 64)
_OFF2 = (96, 232)


def _build_tidx(nid3, neighbors3):
    tcol = jnp.arange(_NTYPE, dtype=jnp.int32)
    z = lambda k: jnp.zeros((k,), jnp.int32)
    idx0, idx1, idx2 = [], [], []
    for s in range(_NSCHEMA):
        toff = tcol * 2 + s
        idx0.append((nid3[:, None] * 6 + toff[None, :]).reshape(-1))
        x1 = neighbors3[:, :, s * _TOT: s * _TOT + 3]
        idx1.append((x1 * 6 + toff[None, :, None]).reshape(-1))
        x2 = neighbors3[:, :, s * _TOT + 3: (s + 1) * _TOT]
        idx2.append((x2 * 6 + toff[None, :, None]).reshape(-1))
    return jnp.concatenate([
        idx0[0], z(7), idx0[1], z(7),
        idx1[0], z(5), idx1[1], z(5),
        idx2[0], z(1), idx2[1], z(145),
    ])


def _build_gidx(nid, neighbors3):
    tidx = _build_tidx(nid[:_NROWS], neighbors3)
    return jnp.concatenate(
        [nid.reshape(_NW, _RPW), tidx.reshape(_NW, _RPW)], axis=1).reshape(-1)


def _sc_gather(btab, ttab, gidx):
    mesh = plsc.VectorSubcoreMesh(core_axis_name="c", subcore_axis_name="s")

    @functools.partial(
        pl.kernel,
        mesh=mesh,
        out_type=[jax.ShapeDtypeStruct((_B, 256), jnp.float32),
                  jax.ShapeDtypeStruct((_B, _ED), jnp.float32)],
        scratch_types=[pltpu.VMEM((_RPW,), jnp.int32),
                       pltpu.VMEM((_RPW,), jnp.int32),
                       pltpu.VMEM((_RPW, 256), jnp.float32),
                       pltpu.VMEM((_RPW, _ED), jnp.float32),
                       pltpu.SemaphoreType.DMA,
                       pltpu.SemaphoreType.DMA],
    )
    def k(btab_h, ttab_h, gidx_h, bout, tout, gvb, gvt, brv, trv, s1, s2):
        wid = lax.axis_index("s") * _NC + lax.axis_index("c")
        g0 = wid * 2 * _RPW
        ia = pltpu.async_copy(gidx_h.at[pl.ds(g0, _RPW)], gvb, s1)
        ib = pltpu.async_copy(gidx_h.at[pl.ds(g0 + _RPW, _RPW)], gvt, s2)
        ia.wait()
        ib.wait()
        ga = pltpu.async_copy(btab_h.at[gvb], brv, s1)
        gb = pltpu.async_copy(ttab_h.at[gvt], trv, s2)
        ga.wait()
        sa = pltpu.async_copy(brv, bout.at[pl.ds(wid * _RPW, _RPW)], s1)
        gb.wait()
        sb = pltpu.async_copy(trv, tout.at[pl.ds(wid * _RPW, _RPW)], s2)
        sa.wait()
        sb.wait()

    return k(btab, ttab, gidx)


def _mmT(x, w):
    return lax.dot_general(x, w, (((1,), (1,)), ((), ())),
                           preferred_element_type=jnp.float32)


def _mm(x, w):
    return lax.dot_general(x, w, (((1,), (0,)), ((), ())),
                           preferred_element_type=jnp.float32)


def _mm0(x, w):
    return lax.dot_general(x, w, (((0,), (0,)), ((), ())),
                           preferred_element_type=jnp.float32)


def _layer_norm(x, g, b):
    mu = jnp.mean(x, axis=-1, keepdims=True)
    var = jnp.mean((x - mu) * (x - mu), axis=-1, keepdims=True)
    return (x - mu) / jnp.sqrt(var + 1e-6) * g + b


def _seg_mean_mat(groups, size):
    ii = lax.broadcasted_iota(jnp.int32, (groups, groups * size), 0)
    jj = lax.broadcasted_iota(jnp.int32, (groups, groups * size), 1)
    return jnp.where(jj // size == ii, np.float32(1.0 / size), np.float32(0.0))


def _masked_mha(x, wq, wk, wv, wfc, g, b, block, period):
    n = x.shape[0]
    q = _mmT(_layer_norm(x, g, b), wq) * np.float32(1.0 / np.sqrt(_ED))
    k = _mmT(x, wk)
    v = _mmT(x, wv)
    logits = _mmT(q, k)
    ii = lax.broadcasted_iota(jnp.int32, (n, n), 0)
    jj = lax.broadcasted_iota(jnp.int32, (n, n), 1)
    same = (ii // block == jj // block) if block > 1 \
        else (ii % period == jj % period)
    logits = jnp.where(same, logits, np.float32(-1e30))
    mx = jnp.max(logits, axis=1, keepdims=True)
    e = jnp.exp(logits - mx)
    a = e / jnp.sum(e, axis=1, keepdims=True)
    return _mmT(_mm(a, v), wfc) + x


def _dense_body(trows_ref, brows_ref, et_ref, reflect_ref, aws_ref, abs_ref,
                awn_ref, abn_ref, vwq_ref, vwk_ref, vwv_ref, vwfc_ref,
                vlng_ref, vlnb_ref, mwq_ref, mwk_ref, mwv_ref, mwfc_ref,
                mlng_ref, mlnb_ref, out_ref):
    relu = lambda x: jnp.maximum(x, np.float32(0.0))
    trows = trows_ref[...]
    m5 = _seg_mean_mat(27, 5)
    m3 = _seg_mean_mat(9, 3)

    spec = []
    for s in range(_NSCHEMA):
        x0 = trows[_OFF0[s]:_OFF0[s] + 9]
        x1 = trows[_OFF1[s]:_OFF1[s] + 27]
        x2 = trows[_OFF2[s]:_OFF2[s] + 135]
        ws0, ws1 = aws_ref[s, 0], aws_ref[s, 1]
        bs0, bs1 = abs_ref[s, 0], abs_ref[s, 1]
        wn0, wn1 = awn_ref[s, 0], awn_ref[s, 1]
        bn0, bn1 = abn_ref[s, 0], abn_ref[s, 1]
        g1 = relu(jnp.concatenate(
            [_mmT(x1, ws0) + bs0, _mmT(_mm(m5, x2), wn0) + bn0], axis=1))
        g0 = relu(jnp.concatenate(
            [_mmT(x0, ws0) + bs0, _mmT(_mm(m3, x1), wn0) + bn0], axis=1))
        zo = relu(jnp.concatenate(
            [_mmT(g0, ws1) + bs1, _mmT(_mm(m3, g1), wn1) + bn1], axis=1))
        spec.append(_masked_mha(zo, vwq_ref[...], vwk_ref[...], vwv_ref[...],
                                vwfc_ref[...], vlng_ref[...], vlnb_ref[...],
                                block=3, period=1))

    z = jnp.concatenate(spec, axis=0)
    z2 = _masked_mha(z, mwq_ref[...], mwk_ref[...], mwv_ref[...],
                     mwfc_ref[...], mlng_ref[...], mlnb_ref[...],
                     block=1, period=9)

    pi = lax.broadcasted_iota(jnp.int32, (9, 18), 0)
    pj = lax.broadcasted_iota(jnp.int32, (9, 18), 1)
    mpool = jnp.where((pj % 9) == pi, np.float32(0.5), np.float32(0.0))
    pooled = _mm(mpool, z2)

    tbl = jnp.zeros((9, 256), jnp.float32)
    rt = lax.broadcasted_iota(jnp.int32, (9, 1), 0)
    for t in range(_NTYPE):
        m_t = jnp.where(rt % 3 == t, np.float32(1.0), np.float32(0.0))
        tbl = tbl + _mm(pooled * m_t, reflect_ref[t])

    keyl = et_ref[0:1, :] * 3 + et_ref[1:2, :]
    i9 = lax.broadcasted_iota(jnp.int32, (9, _B), 0)
    oht = jnp.where(keyl == i9, np.float32(1.0), np.float32(0.0))
    res = brows_ref[...] + _mm0(oht, tbl)
    nrm = jnp.maximum(jnp.sqrt(jnp.sum(res * res, axis=1, keepdims=True)),
                      np.float32(1e-12))
    out_ref[...] = res / nrm


def _dense(trows, brows, et, reflect, aws, ab_s, awn, abn, vwq, vwk, vwv,
           vwfc, vlng, vlnb, mwq, mwk, mwv, mwfc, mlng, mlnb):
    return pl.pallas_call(
        _dense_body,
        out_shape=jax.ShapeDtypeStruct((_B, 256), jnp.float32),
    )(trows, brows, et, reflect, aws, ab_s, awn, abn, vwq, vwk, vwv,
      vwfc, vlng, vlnb, mwq, mwk, mwv, mwfc, mlng, mlnb)


def kernel(base_embed_w, type_embed, reflect, agg_w_self, agg_b_self,
           agg_w_neigh, agg_b_neigh, vw_q, vw_k, vw_v, vw_fc, vln_g, vln_b,
           mw_q, mw_k, mw_v, mw_fc, mln_g, mln_b, nodeids, edgetype,
           neighbors):
    nid = nodeids.astype(jnp.int32)
    gidx = _build_gidx(nid, neighbors[:_NROWS].astype(jnp.int32))
    bout, tout = _sc_gather(base_embed_w, type_embed.reshape(-1, _ED), gidx)
    return _dense(tout, bout, edgetype.astype(jnp.int32), reflect,
                  agg_w_self, agg_b_self, agg_w_neigh, agg_b_neigh, vw_q,
                  vw_k, vw_v, vw_fc, vln_g, vln_b, mw_q, mw_k, mw_v, mw_fc,
                  mln_g, mln_b)


# speedup vs baseline: 1.0652x; 1.0652x over previous
import functools

import jax
import jax.numpy as jnp
import numpy as np
from jax import lax
from jax.experimental import pallas as pl
from jax.experimental.pallas import tpu as pltpu
from jax.experimental.pallas import tpu_sc as plsc

_B = 512
_ED = 128
_NTYPE = 3
_NSCHEMA = 2
_TOT = 18
_NROWS = 3

_NC = 2
_NS = 16
_NW = _NC * _NS
_RPW = _B // _NW

_OFF0 = (0, 16)
_OFF1 = (32, 64)
_OFF2 = (96, 232)


_AOFF = (np.arange(_NTYPE)[None, :, None] * 2 +
         np.arange(2 * _TOT)[None, None, :] // _TOT +
         np.zeros((_NROWS, 1, 1), np.int64)).astype(np.int32)


def _build_tidx(nid3, neighbors3):
    tcol = jnp.arange(_NTYPE, dtype=jnp.int32)
    z = lambda k: jnp.zeros((k,), jnp.int32)
    nbidx = neighbors3 * 6 + jnp.asarray(_AOFF)
    idx0, idx1, idx2 = [], [], []
    for s in range(_NSCHEMA):
        toff = tcol * 2 + s
        idx0.append((nid3[:, None] * 6 + toff[None, :]).reshape(-1))
        idx1.append(nbidx[:, :, s * _TOT: s * _TOT + 3].reshape(-1))
        idx2.append(nbidx[:, :, s * _TOT + 3: (s + 1) * _TOT].reshape(-1))
    return jnp.concatenate([
        idx0[0], z(7), idx0[1], z(7),
        idx1[0], z(5), idx1[1], z(5),
        idx2[0], z(1), idx2[1], z(145),
    ])


def _build_gidx(nid, neighbors3):
    tidx = _build_tidx(nid[:_NROWS], neighbors3)
    return jnp.concatenate(
        [nid.reshape(_NW, _RPW), tidx.reshape(_NW, _RPW)], axis=1).reshape(-1)


def _sc_gather(btab, ttab, gidx):
    mesh = plsc.VectorSubcoreMesh(core_axis_name="c", subcore_axis_name="s")

    @functools.partial(
        pl.kernel,
        mesh=mesh,
        out_type=[jax.ShapeDtypeStruct((_B, 256), jnp.float32),
                  jax.ShapeDtypeStruct((_B, _ED), jnp.float32)],
        scratch_types=[pltpu.VMEM((_RPW,), jnp.int32),
                       pltpu.VMEM((_RPW,), jnp.int32),
                       pltpu.VMEM((_RPW, 256), jnp.float32),
                       pltpu.VMEM((_RPW, _ED), jnp.float32),
                       pltpu.SemaphoreType.DMA,
                       pltpu.SemaphoreType.DMA],
    )
    def k(btab_h, ttab_h, gidx_h, bout, tout, gvb, gvt, brv, trv, s1, s2):
        wid = lax.axis_index("s") * _NC + lax.axis_index("c")
        g0 = wid * 2 * _RPW
        ia = pltpu.async_copy(gidx_h.at[pl.ds(g0, _RPW)], gvb, s1)
        ib = pltpu.async_copy(gidx_h.at[pl.ds(g0 + _RPW, _RPW)], gvt, s2)
        ia.wait()
        ib.wait()
        ga = pltpu.async_copy(btab_h.at[gvb], brv, s1)
        gb = pltpu.async_copy(ttab_h.at[gvt], trv, s2)
        ga.wait()
        sa = pltpu.async_copy(brv, bout.at[pl.ds(wid * _RPW, _RPW)], s1)
        gb.wait()
        sb = pltpu.async_copy(trv, tout.at[pl.ds(wid * _RPW, _RPW)], s2)
        sa.wait()
        sb.wait()

    return k(btab, ttab, gidx)


def _mmT(x, w):
    return lax.dot_general(x, w, (((1,), (1,)), ((), ())),
                           preferred_element_type=jnp.float32)


def _mm(x, w):
    return lax.dot_general(x, w, (((1,), (0,)), ((), ())),
                           preferred_element_type=jnp.float32)


def _mm0(x, w):
    return lax.dot_general(x, w, (((0,), (0,)), ((), ())),
                           preferred_element_type=jnp.float32)


def _layer_norm(x, g, b):
    mu = jnp.mean(x, axis=-1, keepdims=True)
    var = jnp.mean((x - mu) * (x - mu), axis=-1, keepdims=True)
    return (x - mu) / jnp.sqrt(var + 1e-6) * g + b


def _seg_mean_mat(groups, size):
    ii = lax.broadcasted_iota(jnp.int32, (groups, groups * size), 0)
    jj = lax.broadcasted_iota(jnp.int32, (groups, groups * size), 1)
    return jnp.where(jj // size == ii, np.float32(1.0 / size), np.float32(0.0))


def _masked_mha(x, wq, wk, wv, wfc, g, b, block, period):
    n = x.shape[0]
    q = _mmT(_layer_norm(x, g, b), wq) * np.float32(1.0 / np.sqrt(_ED))
    k = _mmT(x, wk)
    v = _mmT(x, wv)
    logits = _mmT(q, k)
    ii = lax.broadcasted_iota(jnp.int32, (n, n), 0)
    jj = lax.broadcasted_iota(jnp.int32, (n, n), 1)
    same = (ii // block == jj // block) if block > 1 \
        else (ii % period == jj % period)
    logits = jnp.where(same, logits, np.float32(-1e30))
    mx = jnp.max(logits, axis=1, keepdims=True)
    e = jnp.exp(logits - mx)
    a = e / jnp.sum(e, axis=1, keepdims=True)
    return _mmT(_mm(a, v), wfc) + x


def _dense_body(trows_ref, brows_ref, et_ref, reflect_ref, aws_ref, abs_ref,
                awn_ref, abn_ref, vwq_ref, vwk_ref, vwv_ref, vwfc_ref,
                vlng_ref, vlnb_ref, mwq_ref, mwk_ref, mwv_ref, mwfc_ref,
                mlng_ref, mlnb_ref, out_ref):
    relu = lambda x: jnp.maximum(x, np.float32(0.0))
    trows = trows_ref[...]
    m5 = _seg_mean_mat(54, 5)
    m3 = _seg_mean_mat(18, 3)

    x0 = jnp.concatenate([trows[_OFF0[0]:_OFF0[0] + 9],
                          trows[_OFF0[1]:_OFF0[1] + 9]], axis=0)
    x1 = jnp.concatenate([trows[_OFF1[0]:_OFF1[0] + 27],
                          trows[_OFF1[1]:_OFF1[1] + 27]], axis=0)
    x2 = jnp.concatenate([trows[_OFF2[0]:_OFF2[0] + 135],
                          trows[_OFF2[1]:_OFF2[1] + 135]], axis=0)

    w0 = jnp.concatenate([aws_ref[0, 0], aws_ref[1, 0]], axis=0)
    w1 = jnp.concatenate([aws_ref[0, 1], aws_ref[1, 1]], axis=0)
    wn0 = jnp.concatenate([awn_ref[0, 0], awn_ref[1, 0]], axis=0)
    wn1 = jnp.concatenate([awn_ref[0, 1], awn_ref[1, 1]], axis=0)

    def half_mask(n):
        r = lax.broadcasted_iota(jnp.int32, (n, 1), 0)
        return jnp.where(r < n // 2, np.float32(1.0), np.float32(0.0))

    def blend(y, m):
        return y[:, 0:64] * m + y[:, 64:128] * (np.float32(1.0) - m)

    def bias(bref, l, m):
        return bref[0, l] * m + bref[1, l] * (np.float32(1.0) - m)

    m54, m18 = half_mask(54), half_mask(18)
    bs0_54, bn0_54 = bias(abs_ref, 0, m54), bias(abn_ref, 0, m54)
    bs0_18, bn0_18 = bias(abs_ref, 0, m18), bias(abn_ref, 0, m18)
    bs1_18, bn1_18 = bias(abs_ref, 1, m18), bias(abn_ref, 1, m18)

    g1 = relu(jnp.concatenate(
        [blend(_mmT(x1, w0), m54) + bs0_54,
         blend(_mmT(_mm(m5, x2), wn0), m54) + bn0_54], axis=1))
    g0 = relu(jnp.concatenate(
        [blend(_mmT(x0, w0), m18) + bs0_18,
         blend(_mmT(_mm(m3, x1), wn0), m18) + bn0_18], axis=1))
    zo = relu(jnp.concatenate(
        [blend(_mmT(g0, w1), m18) + bs1_18,
         blend(_mmT(_mm(m3, g1), wn1), m18) + bn1_18], axis=1))

    z = _masked_mha(zo, vwq_ref[...], vwk_ref[...], vwv_ref[...],
                    vwfc_ref[...], vlng_ref[...], vlnb_ref[...],
                    block=3, period=1)
    z2 = _masked_mha(z, mwq_ref[...], mwk_ref[...], mwv_ref[...],
                     mwfc_ref[...], mlng_ref[...], mlnb_ref[...],
                     block=1, period=9)

    pi = lax.broadcasted_iota(jnp.int32, (9, 18), 0)
    pj = lax.broadcasted_iota(jnp.int32, (9, 18), 1)
    mpool = jnp.where((pj % 9) == pi, np.float32(0.5), np.float32(0.0))
    pooled = _mm(mpool, z2)

    tbl = jnp.zeros((9, 256), jnp.float32)
    rt = lax.broadcasted_iota(jnp.int32, (9, 1), 0)
    for t in range(_NTYPE):
        m_t = jnp.where(rt % 3 == t, np.float32(1.0), np.float32(0.0))
        tbl = tbl + _mm(pooled * m_t, reflect_ref[t])

    keyl = et_ref[0:1, :] * 3 + et_ref[1:2, :]
    i9 = lax.broadcasted_iota(jnp.int32, (9, _B), 0)
    oht = jnp.where(keyl == i9, np.float32(1.0), np.float32(0.0))
    res = brows_ref[...] + _mm0(oht, tbl)
    nrm = jnp.maximum(jnp.sqrt(jnp.sum(res * res, axis=1, keepdims=True)),
                      np.float32(1e-12))
    out_ref[...] = res / nrm


def _dense(trows, brows, et, reflect, aws, ab_s, awn, abn, vwq, vwk, vwv,
           vwfc, vlng, vlnb, mwq, mwk, mwv, mwfc, mlng, mlnb):
    return pl.pallas_call(
        _dense_body,
        out_shape=jax.ShapeDtypeStruct((_B, 256), jnp.float32),
    )(trows, brows, et, reflect, aws, ab_s, awn, abn, vwq, vwk, vwv,
      vwfc, vlng, vlnb, mwq, mwk, mwv, mwfc, mlng, mlnb)


def kernel(base_embed_w, type_embed, reflect, agg_w_self, agg_b_self,
           agg_w_neigh, agg_b_neigh, vw_q, vw_k, vw_v, vw_fc, vln_g, vln_b,
           mw_q, mw_k, mw_v, mw_fc, mln_g, mln_b, nodeids, edgetype,
           neighbors):
    nid = nodeids.astype(jnp.int32)
    gidx = _build_gidx(nid, neighbors[:_NROWS].astype(jnp.int32))
    bout, tout = _sc_gather(base_embed_w, type_embed.reshape(-1, _ED), gidx)
    return _dense(tout, bout, edgetype.astype(jnp.int32), reflect,
                  agg_w_self, agg_b_self, agg_w_neigh, agg_b_neigh, vw_q,
                  vw_k, vw_v, vw_fc, vln_g, vln_b, mw_q, mw_k, mw_v, mw_fc,
                  mln_g, mln_b)


# speedup vs baseline: 1.0925x; 1.0255x over previous
import functools

import jax
import jax.numpy as jnp
import numpy as np
from jax import lax
from jax.experimental import pallas as pl
from jax.experimental.pallas import tpu as pltpu
from jax.experimental.pallas import tpu_sc as plsc

_B = 512
_ED = 128
_NTYPE = 3
_NSCHEMA = 2
_TOT = 18
_NROWS = 3

_NC = 2
_NS = 16
_NW = _NC * _NS
_RPW = _B // _NW

_OFF0 = (0, 16)
_OFF1 = (32, 64)
_OFF2 = (96, 232)


_AOFF = (np.arange(_NTYPE)[None, :, None] * 2 +
         np.arange(2 * _TOT)[None, None, :] // _TOT +
         np.zeros((_NROWS, 1, 1), np.int64)).astype(np.int32)


def _build_tidx(nid3, neighbors3):
    tcol = jnp.arange(_NTYPE, dtype=jnp.int32)
    z = lambda k: jnp.zeros((k,), jnp.int32)
    nbidx = neighbors3 * 6 + jnp.asarray(_AOFF)
    idx0, idx1, idx2 = [], [], []
    for s in range(_NSCHEMA):
        toff = tcol * 2 + s
        idx0.append((nid3[:, None] * 6 + toff[None, :]).reshape(-1))
        idx1.append(nbidx[:, :, s * _TOT: s * _TOT + 3].reshape(-1))
        idx2.append(nbidx[:, :, s * _TOT + 3: (s + 1) * _TOT].reshape(-1))
    return jnp.concatenate([
        idx0[0], z(7), idx0[1], z(7),
        idx1[0], z(5), idx1[1], z(5),
        idx2[0], z(1), idx2[1], z(145),
    ])


def _sc_gather(btab, ttab, nid, tidx):
    mesh = plsc.VectorSubcoreMesh(core_axis_name="c", subcore_axis_name="s")

    @functools.partial(
        pl.kernel,
        mesh=mesh,
        out_type=[jax.ShapeDtypeStruct((_B, 256), jnp.float32),
                  jax.ShapeDtypeStruct((_B, _ED), jnp.float32)],
        scratch_types=[pltpu.VMEM((_RPW,), jnp.int32),
                       pltpu.VMEM((_RPW,), jnp.int32),
                       pltpu.VMEM((_RPW, 256), jnp.float32),
                       pltpu.VMEM((_RPW, _ED), jnp.float32),
                       pltpu.SemaphoreType.DMA,
                       pltpu.SemaphoreType.DMA],
    )
    def k(btab_h, ttab_h, nid_h, tidx_h, bout, tout, gvb, gvt, brv, trv,
          s1, s2):
        wid = lax.axis_index("s") * _NC + lax.axis_index("c")
        g0 = wid * _RPW
        ia = pltpu.async_copy(nid_h.at[pl.ds(g0, _RPW)], gvb, s1)
        ib = pltpu.async_copy(tidx_h.at[pl.ds(g0, _RPW)], gvt, s2)
        ia.wait()
        ib.wait()
        ga = pltpu.async_copy(btab_h.at[gvb], brv, s1)
        gb = pltpu.async_copy(ttab_h.at[gvt], trv, s2)
        ga.wait()
        sa = pltpu.async_copy(brv, bout.at[pl.ds(wid * _RPW, _RPW)], s1)
        gb.wait()
        sb = pltpu.async_copy(trv, tout.at[pl.ds(wid * _RPW, _RPW)], s2)
        sa.wait()
        sb.wait()

    return k(btab, ttab, nid, tidx)


def _mmT(x, w):
    return lax.dot_general(x, w, (((1,), (1,)), ((), ())),
                           preferred_element_type=jnp.float32)


def _mm(x, w):
    return lax.dot_general(x, w, (((1,), (0,)), ((), ())),
                           preferred_element_type=jnp.float32)


def _mm0(x, w):
    return lax.dot_general(x, w, (((0,), (0,)), ((), ())),
                           preferred_element_type=jnp.float32)


def _layer_norm(x, g, b):
    mu = jnp.mean(x, axis=-1, keepdims=True)
    var = jnp.mean((x - mu) * (x - mu), axis=-1, keepdims=True)
    return (x - mu) / jnp.sqrt(var + 1e-6) * g + b


def _seg_mean_mat(groups, size):
    ii = lax.broadcasted_iota(jnp.int32, (groups, groups * size), 0)
    jj = lax.broadcasted_iota(jnp.int32, (groups, groups * size), 1)
    return jnp.where(jj // size == ii, np.float32(1.0 / size), np.float32(0.0))


def _masked_mha(x, wq, wk, wv, wfc, g, b, block, period):
    n = x.shape[0]
    q = _mmT(_layer_norm(x, g, b), wq) * np.float32(1.0 / np.sqrt(_ED))
    k = _mmT(x, wk)
    v = _mmT(x, wv)
    logits = _mmT(q, k)
    ii = lax.broadcasted_iota(jnp.int32, (n, n), 0)
    jj = lax.broadcasted_iota(jnp.int32, (n, n), 1)
    same = (ii // block == jj // block) if block > 1 \
        else (ii % period == jj % period)
    logits = jnp.where(same, logits, np.float32(-1e30))
    mx = jnp.max(logits, axis=1, keepdims=True)
    e = jnp.exp(logits - mx)
    a = e / jnp.sum(e, axis=1, keepdims=True)
    return _mmT(_mm(a, v), wfc) + x


def _dense_body(trows_ref, brows_ref, et_ref, reflect_ref, aws_ref, abs_ref,
                awn_ref, abn_ref, vwq_ref, vwk_ref, vwv_ref, vwfc_ref,
                vlng_ref, vlnb_ref, mwq_ref, mwk_ref, mwv_ref, mwfc_ref,
                mlng_ref, mlnb_ref, out_ref):
    relu = lambda x: jnp.maximum(x, np.float32(0.0))
    trows = trows_ref[...]
    m5 = _seg_mean_mat(54, 5)
    m3 = _seg_mean_mat(18, 3)

    x0 = jnp.concatenate([trows[_OFF0[0]:_OFF0[0] + 9],
                          trows[_OFF0[1]:_OFF0[1] + 9]], axis=0)
    x1 = jnp.concatenate([trows[_OFF1[0]:_OFF1[0] + 27],
                          trows[_OFF1[1]:_OFF1[1] + 27]], axis=0)
    x2 = jnp.concatenate([trows[_OFF2[0]:_OFF2[0] + 135],
                          trows[_OFF2[1]:_OFF2[1] + 135]], axis=0)

    w0 = jnp.concatenate([aws_ref[0, 0], aws_ref[1, 0]], axis=0)
    w1 = jnp.concatenate([aws_ref[0, 1], aws_ref[1, 1]], axis=0)
    wn0 = jnp.concatenate([awn_ref[0, 0], awn_ref[1, 0]], axis=0)
    wn1 = jnp.concatenate([awn_ref[0, 1], awn_ref[1, 1]], axis=0)

    def half_mask(n):
        r = lax.broadcasted_iota(jnp.int32, (n, 1), 0)
        return jnp.where(r < n // 2, np.float32(1.0), np.float32(0.0))

    def blend(y, m):
        return y[:, 0:64] * m + y[:, 64:128] * (np.float32(1.0) - m)

    def bias(bref, l, m):
        return bref[0, l] * m + bref[1, l] * (np.float32(1.0) - m)

    m54, m18 = half_mask(54), half_mask(18)
    bs0_54, bn0_54 = bias(abs_ref, 0, m54), bias(abn_ref, 0, m54)
    bs0_18, bn0_18 = bias(abs_ref, 0, m18), bias(abn_ref, 0, m18)
    bs1_18, bn1_18 = bias(abs_ref, 1, m18), bias(abn_ref, 1, m18)

    g1 = relu(jnp.concatenate(
        [blend(_mmT(x1, w0), m54) + bs0_54,
         blend(_mmT(_mm(m5, x2), wn0), m54) + bn0_54], axis=1))
    g0 = relu(jnp.concatenate(
        [blend(_mmT(x0, w0), m18) + bs0_18,
         blend(_mmT(_mm(m3, x1), wn0), m18) + bn0_18], axis=1))
    zo = relu(jnp.concatenate(
        [blend(_mmT(g0, w1), m18) + bs1_18,
         blend(_mmT(_mm(m3, g1), wn1), m18) + bn1_18], axis=1))

    z = _masked_mha(zo, vwq_ref[...], vwk_ref[...], vwv_ref[...],
                    vwfc_ref[...], vlng_ref[...], vlnb_ref[...],
                    block=3, period=1)
    z2 = _masked_mha(z, mwq_ref[...], mwk_ref[...], mwv_ref[...],
                     mwfc_ref[...], mlng_ref[...], mlnb_ref[...],
                     block=1, period=9)

    pi = lax.broadcasted_iota(jnp.int32, (9, 18), 0)
    pj = lax.broadcasted_iota(jnp.int32, (9, 18), 1)
    mpool = jnp.where((pj % 9) == pi, np.float32(0.5), np.float32(0.0))
    pooled = _mm(mpool, z2)

    tbl = jnp.zeros((9, 256), jnp.float32)
    rt = lax.broadcasted_iota(jnp.int32, (9, 1), 0)
    for t in range(_NTYPE):
        m_t = jnp.where(rt % 3 == t, np.float32(1.0), np.float32(0.0))
        tbl = tbl + _mm(pooled * m_t, reflect_ref[t])

    keyl = et_ref[0:1, :] * 3 + et_ref[1:2, :]
    i9 = lax.broadcasted_iota(jnp.int32, (9, _B), 0)
    oht = jnp.where(keyl == i9, np.float32(1.0), np.float32(0.0))
    res = brows_ref[...] + _mm0(oht, tbl)
    nrm = jnp.maximum(jnp.sqrt(jnp.sum(res * res, axis=1, keepdims=True)),
                      np.float32(1e-12))
    out_ref[...] = res / nrm


def _dense(trows, brows, et, reflect, aws, ab_s, awn, abn, vwq, vwk, vwv,
           vwfc, vlng, vlnb, mwq, mwk, mwv, mwfc, mlng, mlnb):
    return pl.pallas_call(
        _dense_body,
        out_shape=jax.ShapeDtypeStruct((_B, 256), jnp.float32),
    )(trows, brows, et, reflect, aws, ab_s, awn, abn, vwq, vwk, vwv,
      vwfc, vlng, vlnb, mwq, mwk, mwv, mwfc, mlng, mlnb)


def kernel(base_embed_w, type_embed, reflect, agg_w_self, agg_b_self,
           agg_w_neigh, agg_b_neigh, vw_q, vw_k, vw_v, vw_fc, vln_g, vln_b,
           mw_q, mw_k, mw_v, mw_fc, mln_g, mln_b, nodeids, edgetype,
           neighbors):
    nid = nodeids.astype(jnp.int32)
    tidx = _build_tidx(nid[:_NROWS], neighbors[:_NROWS].astype(jnp.int32))
    bout, tout = _sc_gather(base_embed_w, type_embed.reshape(-1, _ED),
                            nid, tidx)
    return _dense(tout, bout, edgetype.astype(jnp.int32), reflect,
                  agg_w_self, agg_b_self, agg_w_neigh, agg_b_neigh, vw_q,
                  vw_k, vw_v, vw_fc, vln_g, vln_b, mw_q, mw_k, mw_v, mw_fc,
                  mln_g, mln_b)


# speedup vs baseline: 1.0931x; 1.0006x over previous
import functools

import jax
import jax.numpy as jnp
import numpy as np
from jax import lax
from jax.experimental import pallas as pl
from jax.experimental.pallas import tpu as pltpu
from jax.experimental.pallas import tpu_sc as plsc

_B = 512
_ED = 128
_NTYPE = 3
_NSCHEMA = 2
_TOT = 18
_NROWS = 3

_NC = 2
_NS = 16
_NW = _NC * _NS
_RPW = _B // _NW

_OFF0 = (0, 16)
_OFF1 = (32, 64)
_OFF2 = (96, 232)


_AOFF = (np.arange(_NTYPE)[None, :, None] * 2 +
         np.arange(2 * _TOT)[None, None, :] // _TOT +
         np.zeros((_NROWS, 1, 1), np.int64)).astype(np.int32)


def _build_tidx(nid3, neighbors3):
    tcol = jnp.arange(_NTYPE, dtype=jnp.int32)
    z = lambda k: jnp.zeros((k,), jnp.int32)
    nbidx = neighbors3 * 6 + jnp.asarray(_AOFF)
    idx0, idx1, idx2 = [], [], []
    for s in range(_NSCHEMA):
        toff = tcol * 2 + s
        idx0.append((nid3[:, None] * 6 + toff[None, :]).reshape(-1))
        idx1.append(nbidx[:, :, s * _TOT: s * _TOT + 3].reshape(-1))
        idx2.append(nbidx[:, :, s * _TOT + 3: (s + 1) * _TOT].reshape(-1))
    return jnp.concatenate([
        idx0[0], z(7), idx0[1], z(7),
        idx1[0], z(5), idx1[1], z(5),
        idx2[0], z(1), idx2[1], z(145),
    ])


def _sc_gather(btab, ttab, nid, tidx):
    mesh = plsc.VectorSubcoreMesh(core_axis_name="c", subcore_axis_name="s")

    @functools.partial(
        pl.kernel,
        mesh=mesh,
        out_type=[jax.ShapeDtypeStruct((_B, 256), jnp.float32),
                  jax.ShapeDtypeStruct((_B, _ED), jnp.float32)],
        scratch_types=[pltpu.VMEM((_RPW,), jnp.int32),
                       pltpu.VMEM((_RPW,), jnp.int32),
                       pltpu.VMEM((_RPW, 256), jnp.float32),
                       pltpu.VMEM((_RPW, _ED), jnp.float32),
                       pltpu.SemaphoreType.DMA,
                       pltpu.SemaphoreType.DMA],
    )
    def k(btab_h, ttab_h, nid_h, tidx_h, bout, tout, gvb, gvt, brv, trv,
          s1, s2):
        wid = lax.axis_index("s") * _NC + lax.axis_index("c")
        g0 = wid * _RPW
        ia = pltpu.async_copy(nid_h.at[pl.ds(g0, _RPW)], gvb, s1)
        ib = pltpu.async_copy(tidx_h.at[pl.ds(g0, _RPW)], gvt, s2)
        ia.wait()
        ga = pltpu.async_copy(btab_h.at[gvb], brv, s1)
        ib.wait()
        gb = pltpu.async_copy(ttab_h.at[gvt], trv, s2)
        ga.wait()
        sa = pltpu.async_copy(brv, bout.at[pl.ds(wid * _RPW, _RPW)], s1)
        gb.wait()
        sb = pltpu.async_copy(trv, tout.at[pl.ds(wid * _RPW, _RPW)], s2)
        sa.wait()
        sb.wait()

    return k(btab, ttab, nid, tidx)


def _mmT(x, w):
    return lax.dot_general(x, w, (((1,), (1,)), ((), ())),
                           preferred_element_type=jnp.float32)


def _mm(x, w):
    return lax.dot_general(x, w, (((1,), (0,)), ((), ())),
                           preferred_element_type=jnp.float32)


def _mm0(x, w):
    return lax.dot_general(x, w, (((0,), (0,)), ((), ())),
                           preferred_element_type=jnp.float32)


def _layer_norm(x, g, b):
    mu = jnp.mean(x, axis=-1, keepdims=True)
    var = jnp.mean((x - mu) * (x - mu), axis=-1, keepdims=True)
    return (x - mu) / jnp.sqrt(var + 1e-6) * g + b


def _seg_mean_mat(groups, size):
    ii = lax.broadcasted_iota(jnp.int32, (groups, groups * size), 0)
    jj = lax.broadcasted_iota(jnp.int32, (groups, groups * size), 1)
    return jnp.where(jj // size == ii, np.float32(1.0 / size), np.float32(0.0))


def _masked_mha(x, wq, wk, wv, wfc, g, b, block, period):
    n = x.shape[0]
    q = _mmT(_layer_norm(x, g, b), wq) * np.float32(1.0 / np.sqrt(_ED))
    k = _mmT(x, wk)
    v = _mmT(x, wv)
    logits = _mmT(q, k)
    ii = lax.broadcasted_iota(jnp.int32, (n, n), 0)
    jj = lax.broadcasted_iota(jnp.int32, (n, n), 1)
    same = (ii // block == jj // block) if block > 1 \
        else (ii % period == jj % period)
    logits = jnp.where(same, logits, np.float32(-1e30))
    mx = jnp.max(logits, axis=1, keepdims=True)
    e = jnp.exp(logits - mx)
    a = e / jnp.sum(e, axis=1, keepdims=True)
    return _mmT(_mm(a, v), wfc) + x


def _dense_body(trows_ref, brows_ref, et_ref, reflect_ref, aws_ref, abs_ref,
                awn_ref, abn_ref, vwq_ref, vwk_ref, vwv_ref, vwfc_ref,
                vlng_ref, vlnb_ref, mwq_ref, mwk_ref, mwv_ref, mwfc_ref,
                mlng_ref, mlnb_ref, out_ref):
    relu = lambda x: jnp.maximum(x, np.float32(0.0))
    trows = trows_ref[...]
    m5 = _seg_mean_mat(54, 5)
    m3 = _seg_mean_mat(18, 3)

    x0 = jnp.concatenate([trows[_OFF0[0]:_OFF0[0] + 9],
                          trows[_OFF0[1]:_OFF0[1] + 9]], axis=0)
    x1 = jnp.concatenate([trows[_OFF1[0]:_OFF1[0] + 27],
                          trows[_OFF1[1]:_OFF1[1] + 27]], axis=0)
    x2 = jnp.concatenate([trows[_OFF2[0]:_OFF2[0] + 135],
                          trows[_OFF2[1]:_OFF2[1] + 135]], axis=0)

    w0 = jnp.concatenate([aws_ref[0, 0], aws_ref[1, 0]], axis=0)
    w1 = jnp.concatenate([aws_ref[0, 1], aws_ref[1, 1]], axis=0)
    wn0 = jnp.concatenate([awn_ref[0, 0], awn_ref[1, 0]], axis=0)
    wn1 = jnp.concatenate([awn_ref[0, 1], awn_ref[1, 1]], axis=0)

    def half_mask(n):
        r = lax.broadcasted_iota(jnp.int32, (n, 1), 0)
        return jnp.where(r < n // 2, np.float32(1.0), np.float32(0.0))

    def blend(y, m):
        return y[:, 0:64] * m + y[:, 64:128] * (np.float32(1.0) - m)

    def bias(bref, l, m):
        return bref[0, l] * m + bref[1, l] * (np.float32(1.0) - m)

    m54, m18 = half_mask(54), half_mask(18)
    bs0_54, bn0_54 = bias(abs_ref, 0, m54), bias(abn_ref, 0, m54)
    bs0_18, bn0_18 = bias(abs_ref, 0, m18), bias(abn_ref, 0, m18)
    bs1_18, bn1_18 = bias(abs_ref, 1, m18), bias(abn_ref, 1, m18)

    g1 = relu(jnp.concatenate(
        [blend(_mmT(x1, w0), m54) + bs0_54,
         blend(_mmT(_mm(m5, x2), wn0), m54) + bn0_54], axis=1))
    g0 = relu(jnp.concatenate(
        [blend(_mmT(x0, w0), m18) + bs0_18,
         blend(_mmT(_mm(m3, x1), wn0), m18) + bn0_18], axis=1))
    zo = relu(jnp.concatenate(
        [blend(_mmT(g0, w1), m18) + bs1_18,
         blend(_mmT(_mm(m3, g1), wn1), m18) + bn1_18], axis=1))

    z = _masked_mha(zo, vwq_ref[...], vwk_ref[...], vwv_ref[...],
                    vwfc_ref[...], vlng_ref[...], vlnb_ref[...],
                    block=3, period=1)
    z2 = _masked_mha(z, mwq_ref[...], mwk_ref[...], mwv_ref[...],
                     mwfc_ref[...], mlng_ref[...], mlnb_ref[...],
                     block=1, period=9)

    pi = lax.broadcasted_iota(jnp.int32, (9, 18), 0)
    pj = lax.broadcasted_iota(jnp.int32, (9, 18), 1)
    mpool = jnp.where((pj % 9) == pi, np.float32(0.5), np.float32(0.0))
    pooled = _mm(mpool, z2)

    tbl = jnp.zeros((9, 256), jnp.float32)
    rt = lax.broadcasted_iota(jnp.int32, (9, 1), 0)
    for t in range(_NTYPE):
        m_t = jnp.where(rt % 3 == t, np.float32(1.0), np.float32(0.0))
        tbl = tbl + _mm(pooled * m_t, reflect_ref[t])

    keyl = et_ref[0:1, :] * 3 + et_ref[1:2, :]
    i9 = lax.broadcasted_iota(jnp.int32, (9, _B), 0)
    oht = jnp.where(keyl == i9, np.float32(1.0), np.float32(0.0))
    res = brows_ref[...] + _mm0(oht, tbl)
    nrm = jnp.maximum(jnp.sqrt(jnp.sum(res * res, axis=1, keepdims=True)),
                      np.float32(1e-12))
    out_ref[...] = res / nrm


def _dense(trows, brows, et, reflect, aws, ab_s, awn, abn, vwq, vwk, vwv,
           vwfc, vlng, vlnb, mwq, mwk, mwv, mwfc, mlng, mlnb):
    return pl.pallas_call(
        _dense_body,
        out_shape=jax.ShapeDtypeStruct((_B, 256), jnp.float32),
    )(trows, brows, et, reflect, aws, ab_s, awn, abn, vwq, vwk, vwv,
      vwfc, vlng, vlnb, mwq, mwk, mwv, mwfc, mlng, mlnb)


def kernel(base_embed_w, type_embed, reflect, agg_w_self, agg_b_self,
           agg_w_neigh, agg_b_neigh, vw_q, vw_k, vw_v, vw_fc, vln_g, vln_b,
           mw_q, mw_k, mw_v, mw_fc, mln_g, mln_b, nodeids, edgetype,
           neighbors):
    nid = nodeids.astype(jnp.int32)
    tidx = _build_tidx(nid[:_NROWS], neighbors[:_NROWS].astype(jnp.int32))
    bout, tout = _sc_gather(base_embed_w, type_embed.reshape(-1, _ED),
                            nid, tidx)
    return _dense(tout, bout, edgetype.astype(jnp.int32), reflect,
                  agg_w_self, agg_b_self, agg_w_neigh, agg_b_neigh, vw_q,
                  vw_k, vw_v, vw_fc, vln_g, vln_b, mw_q, mw_k, mw_v, mw_fc,
                  mln_g, mln_b)
